# Initial kernel scaffold; baseline (speedup 1.0000x reference)
import jax
import jax.numpy as jnp
from jax.experimental import pallas as pl


def kernel(x, edge_index, edge_attr, batch, descriptors, params):
    raise NotImplementedError("write your pallas kernel here")


# speedup vs baseline: 1.2400x; 1.2400x over previous
import functools

import jax
import jax.numpy as jnp
from jax import lax
from jax.experimental import pallas as pl
from jax.experimental.pallas import tpu as pltpu
from jax.experimental.pallas import tpu_sc as plsc

H = 64
CHUNK = 80


def _make_edge_pass(N, E, want_deg):
  NS = 16
  epert = E // NS
  iters = epert // CHUNK
  assert epert % CHUNK == 0
  rows_main = ((N // NS) + 7) // 8 * 8
  rows_last = N - rows_main * (NS - 1)
  assert rows_last > 0 and rows_last % 8 == 0

  out_type = [jax.ShapeDtypeStruct((2 * N, 32), jnp.float32)]
  if want_deg:
    out_type.append(jax.ShapeDtypeStruct((N, 8), jnp.float32))

  mesh = plsc.VectorSubcoreMesh(core_axis_name="c", subcore_axis_name="s")

  def body(hp_ref, ce_ref, src_ref, dst_ref, z2_ref, z1_ref, ones_ref, *rest):
    if want_deg:
      s_out, deg_out = rest[0], rest[1]
      scr = rest[2:]
    else:
      s_out = rest[0]
      scr = rest[1:]
    (src_v, idx_v, dst_v, ce_v, rows_v, ones_v, S_sh, deg_sh, sem) = scr

    cid = lax.axis_index("c")
    sid = lax.axis_index("s")

    pltpu.sync_copy(ones_ref, ones_v)

    base_r = sid * rows_main

    @pl.when(sid < NS - 1)
    def _():
      pltpu.sync_copy(z2_ref.at[pl.ds(0, rows_main)],
                      S_sh.at[pl.ds(base_r, rows_main)])
      pltpu.sync_copy(z1_ref.at[pl.ds(0, rows_main)],
                      deg_sh.at[pl.ds(base_r, rows_main)])

    @pl.when(sid == NS - 1)
    def _():
      pltpu.sync_copy(z2_ref.at[pl.ds(0, rows_last)],
                      S_sh.at[pl.ds(base_r, rows_last)])
      pltpu.sync_copy(z1_ref.at[pl.ds(0, rows_last)],
                      deg_sh.at[pl.ds(base_r, rows_last)])

    plsc.subcore_barrier()

    tile_base = sid * epert
    col_base = cid * N

    def chunk_body(i, carry):
      base = tile_base + i * CHUNK
      pltpu.sync_copy(src_ref.at[pl.ds(base, CHUNK)], src_v)
      pltpu.sync_copy(dst_ref.at[pl.ds(base, CHUNK)], dst_v)
      pltpu.sync_copy(ce_ref.at[pl.ds(cid * E + base, CHUNK)], ce_v)
      for j in range(CHUNK // 16):
        s = src_v[pl.ds(j * 16, 16)]
        idx_v[pl.ds(j * 16, 16)] = s + col_base
      pltpu.async_copy(hp_ref.at[idx_v], rows_v, sem).wait()

      def row_body(r, c2):
        for half in range(2):
          c0 = half * 16
          v = rows_v[r, pl.ds(c0, 16)] + ce_v[r, pl.ds(c0, 16)]
          rows_v[r, pl.ds(c0, 16)] = jnp.maximum(v, 0.0)
        return c2

      lax.fori_loop(0, CHUNK, row_body, 0)
      pltpu.sync_copy(rows_v, S_sh.at[dst_v], add=True)
      if want_deg:
        @pl.when(cid == 0)
        def _():
          pltpu.sync_copy(ones_v, deg_sh.at[dst_v], add=True)
      return carry

    lax.fori_loop(0, iters, chunk_body, 0)
    plsc.subcore_barrier()

    @pl.when(sid < NS - 1)
    def _():
      pltpu.sync_copy(S_sh.at[pl.ds(base_r, rows_main)],
                      s_out.at[pl.ds(col_base + base_r, rows_main)])

    @pl.when(sid == NS - 1)
    def _():
      pltpu.sync_copy(S_sh.at[pl.ds(base_r, rows_last)],
                      s_out.at[pl.ds(col_base + base_r, rows_last)])

    if want_deg:
      @pl.when((cid == 0) & (sid < NS - 1))
      def _():
        pltpu.sync_copy(deg_sh.at[pl.ds(base_r, rows_main)],
                        deg_out.at[pl.ds(base_r, rows_main)])

      @pl.when((cid == 0) & (sid == NS - 1))
      def _():
        pltpu.sync_copy(deg_sh.at[pl.ds(base_r, rows_last)],
                        deg_out.at[pl.ds(base_r, rows_last)])

  return pl.kernel(
      body,
      mesh=mesh,
      out_type=out_type,
      compiler_params=pltpu.CompilerParams(use_tc_tiling_on_sc=False),
      scratch_types=[
          pltpu.VMEM((CHUNK,), jnp.int32),
          pltpu.VMEM((CHUNK,), jnp.int32),
          pltpu.VMEM((CHUNK,), jnp.int32),
          pltpu.VMEM((CHUNK, 32), jnp.float32),
          pltpu.VMEM((CHUNK, 32), jnp.float32),
          pltpu.VMEM((CHUNK, 8), jnp.float32),
          pltpu.VMEM_SHARED((N, 32), jnp.float32),
          pltpu.VMEM_SHARED((N, 8), jnp.float32),
          pltpu.SemaphoreType.DMA,
      ],
  )


def _encode(x, Wn, bn, A1, BN=2000):
  N = x.shape[0]

  def body(x_ref, wn_ref, bn_ref, a1_ref, h_ref, hp_ref):
    h = jnp.dot(x_ref[...], wn_ref[...],
                preferred_element_type=jnp.float32) + bn_ref[...]
    h_ref[...] = h
    hp = jnp.dot(h, a1_ref[...], preferred_element_type=jnp.float32)
    hp_ref[0] = hp[:, :32]
    hp_ref[1] = hp[:, 32:]

  return pl.pallas_call(
      body,
      grid=(N // BN,),
      in_specs=[
          pl.BlockSpec((BN, x.shape[1]), lambda i: (i, 0)),
          pl.BlockSpec((x.shape[1], H), lambda i: (0, 0)),
          pl.BlockSpec((1, H), lambda i: (0, 0)),
          pl.BlockSpec((H, H), lambda i: (0, 0)),
      ],
      out_specs=[
          pl.BlockSpec((BN, H), lambda i: (i, 0)),
          pl.BlockSpec((2, BN, 32), lambda i: (0, i, 0)),
      ],
      out_shape=[
          jax.ShapeDtypeStruct((N, H), jnp.float32),
          jax.ShapeDtypeStruct((2, N, 32), jnp.float32),
      ],
  )(x, Wn, bn.reshape(1, H), A1)


def _edge_const(edge_attr, Wce, bce, BE=4000):
  E, ED = edge_attr.shape

  def body(ea_ref, w_ref, b_ref, ce_ref):
    z = jnp.dot(ea_ref[...], w_ref[...],
                preferred_element_type=jnp.float32) + b_ref[...]
    for l in range(3):
      for c in range(2):
        ce_ref[l, c] = z[:, l * H + c * 32:l * H + (c + 1) * 32]

  return pl.pallas_call(
      body,
      grid=(E // BE,),
      in_specs=[
          pl.BlockSpec((BE, ED), lambda i: (i, 0)),
          pl.BlockSpec((ED, 3 * H), lambda i: (0, 0)),
          pl.BlockSpec((1, 3 * H), lambda i: (0, 0)),
      ],
      out_specs=pl.BlockSpec((3, 2, BE, 32), lambda i: (0, 0, i, 0)),
      out_shape=jax.ShapeDtypeStruct((3, 2, E, 32), jnp.float32),
  )(edge_attr, Wce, bce.reshape(1, 3 * H))


def _update(h, S2, deg, W1, W2, vrow, bu1, u2w, u2b, Anext, BN=2000):
  N = h.shape[0]
  has_next = Anext is not None

  def body(h_ref, s_ref, deg_ref, w1_ref, w2_ref, v_ref, b1_ref, w3_ref,
           b3_ref, *rest):
    if has_next:
      an_ref, hn_ref, hp_ref = rest
    else:
      (hn_ref,) = rest
    hv = h_ref[...]
    w2 = w2_ref[...]
    u = (jnp.dot(hv, w1_ref[...], preferred_element_type=jnp.float32)
         + jnp.dot(s_ref[0], w2[:32, :], preferred_element_type=jnp.float32)
         + jnp.dot(s_ref[1], w2[32:, :], preferred_element_type=jnp.float32)
         + deg_ref[...][:, :1] * v_ref[...] + b1_ref[...])
    u = jnp.maximum(u, 0.0)
    hn = jnp.dot(u, w3_ref[...], preferred_element_type=jnp.float32) + b3_ref[...]
    hn_ref[...] = hn
    if has_next:
      hp = jnp.dot(hn, an_ref[...], preferred_element_type=jnp.float32)
      hp_ref[0] = hp[:, :32]
      hp_ref[1] = hp[:, 32:]

  full = lambda i: (0, 0)
  in_specs = [
      pl.BlockSpec((BN, H), lambda i: (i, 0)),
      pl.BlockSpec((2, BN, 32), lambda i: (0, i, 0)),
      pl.BlockSpec((BN, 8), lambda i: (i, 0)),
      pl.BlockSpec((H, H), full),
      pl.BlockSpec((H, H), full),
      pl.BlockSpec((1, H), full),
      pl.BlockSpec((1, H), full),
      pl.BlockSpec((H, H), full),
      pl.BlockSpec((1, H), full),
  ]
  out_specs = [pl.BlockSpec((BN, H), lambda i: (i, 0))]
  out_shape = [jax.ShapeDtypeStruct((N, H), jnp.float32)]
  args = [h, S2, deg, W1, W2, vrow.reshape(1, H), bu1.reshape(1, H), u2w,
          u2b.reshape(1, H)]
  if has_next:
    in_specs.append(pl.BlockSpec((H, H), full))
    out_specs.append(pl.BlockSpec((2, BN, 32), lambda i: (0, i, 0)))
    out_shape.append(jax.ShapeDtypeStruct((2, N, 32), jnp.float32))
    args.append(Anext)

  res = pl.pallas_call(
      body,
      grid=(N // BN,),
      in_specs=in_specs,
      out_specs=out_specs,
      out_shape=out_shape,
  )(*args)
  return res if has_next else (res[0], None)


def _pool_readout(h, batch2d, descriptors, r1w, r1b, r2w, r2b, BN=2000):
  N = h.shape[0]
  G, DESC = descriptors.shape
  nblk = N // BN

  def body(h_ref, b_ref, d_ref, r1w_ref, r1b_ref, r2w_ref, r2b_ref, out_ref,
           sums_acc, cnt_acc):
    i = pl.program_id(0)
    onehot = (b_ref[...] == lax.broadcasted_iota(jnp.int32, (1, G), 1)
              ).astype(jnp.float32)
    contrib = lax.dot_general(onehot, h_ref[...], (((0,), (0,)), ((), ())),
                              preferred_element_type=jnp.float32)
    ones_col = jnp.ones((BN, 1), jnp.float32)
    cnt_contrib = lax.dot_general(onehot, ones_col, (((0,), (0,)), ((), ())),
                                  preferred_element_type=jnp.float32)

    @pl.when(i == 0)
    def _():
      sums_acc[...] = contrib
      cnt_acc[...] = cnt_contrib

    @pl.when(i > 0)
    def _():
      sums_acc[...] = sums_acc[...] + contrib
      cnt_acc[...] = cnt_acc[...] + cnt_contrib

    @pl.when(i == nblk - 1)
    def _():
      pooled = sums_acc[...] / jnp.maximum(cnt_acc[...], 1.0)
      r1 = r1w_ref[...]
      z = (jnp.dot(pooled, r1[:H, :], preferred_element_type=jnp.float32)
           + jnp.dot(d_ref[...], r1[H:, :], preferred_element_type=jnp.float32)
           + r1b_ref[...])
      z = jnp.maximum(z, 0.0)
      z2 = jnp.dot(z, r2w_ref[...], preferred_element_type=jnp.float32) \
          + r2b_ref[...]
      out_ref[...] = 1.0 / (1.0 + jnp.exp(-z2))

  full = lambda i: (0, 0)
  return pl.pallas_call(
      body,
      grid=(nblk,),
      in_specs=[
          pl.BlockSpec((BN, H), lambda i: (i, 0)),
          pl.BlockSpec((BN, 1), lambda i: (i, 0)),
          pl.BlockSpec((G, DESC), full),
          pl.BlockSpec((H + DESC, 128), full),
          pl.BlockSpec((1, 128), full),
          pl.BlockSpec((128, 1), full),
          pl.BlockSpec((1, 1), full),
      ],
      out_specs=pl.BlockSpec((G, 1), full),
      out_shape=jax.ShapeDtypeStruct((G, 1), jnp.float32),
      scratch_shapes=[
          pltpu.VMEM((G, H), jnp.float32),
          pltpu.VMEM((G, 1), jnp.float32),
      ],
  )(h, batch2d, descriptors, r1w, r1b.reshape(1, 128), r2w,
    r2b.reshape(1, 1))


def kernel(x, edge_index, edge_attr, batch, descriptors, params):
  N = x.shape[0]
  E = edge_index.shape[1]

  src = edge_index[0].astype(jnp.int32)
  dst = edge_index[1].astype(jnp.int32)
  batch2d = batch.astype(jnp.int32).reshape(N, 1)

  Wn, bn = params["node_enc"]
  We, be = params["edge_enc"]
  layers = params["layers"]

  Bs = [lp["e1"][0][H:] for lp in layers]
  Wce = jnp.concatenate([We @ B for B in Bs], axis=1)
  bce = jnp.concatenate([be @ B + lp["e1"][1]
                         for B, lp in zip(Bs, layers)], axis=0)

  A = [lp["e1"][0][:H] for lp in layers]

  h, hp = _encode(x, Wn, bn, A[0])
  ce = _edge_const(edge_attr, Wce, bce)

  z2 = jnp.zeros((3200, 32), jnp.float32)
  z1 = jnp.zeros((3200, 8), jnp.float32)
  ones_in = jnp.ones((CHUNK, 8), jnp.float32)

  edge_pass_deg = _make_edge_pass(N, E, want_deg=True)
  edge_pass = _make_edge_pass(N, E, want_deg=False)

  deg = None
  for l, lp in enumerate(layers):
    hp2 = hp.reshape(2 * N, 32)
    ce2 = ce[l].reshape(2 * E, 32)
    if l == 0:
      S2, deg = edge_pass_deg(hp2, ce2, src, dst, z2, z1, ones_in)
    else:
      res = edge_pass(hp2, ce2, src, dst, z2, z1, ones_in)
      S2 = res[0] if isinstance(res, (list, tuple)) else res

    u1w, u1b = lp["u1"]
    e2w, e2b = lp["e2"]
    W1 = u1w[:H]
    W2 = e2w @ u1w[H:]
    vrow = e2b @ u1w[H:]
    u2w, u2b = lp["u2"]
    Anext = A[l + 1] if l + 1 < len(layers) else None
    h, hp = _update(h, S2.reshape(2, N, 32), deg, W1, W2, vrow, u1b, u2w,
                    u2b, Anext)

  r1w, r1b = params["r1"]
  r2w, r2b = params["r2"]
  out = _pool_readout(h, batch2d, descriptors, r1w, r1b, r2w, r2b)
  return out.reshape(-1)


# speedup vs baseline: 4.1222x; 3.3242x over previous
import functools

import jax
import jax.numpy as jnp
from jax import lax
from jax.experimental import pallas as pl
from jax.experimental.pallas import tpu as pltpu
from jax.experimental.pallas import tpu_sc as plsc

H = 64
CHUNK = 80


def _make_edge_pass(N, E):
  NS = 16
  epert = E // NS
  iters = epert // CHUNK
  assert epert % CHUNK == 0
  rows_main = ((N // NS) + 7) // 8 * 8
  rows_last = N - rows_main * (NS - 1)
  assert rows_last > 0 and rows_last % 8 == 0

  mesh = plsc.VectorSubcoreMesh(core_axis_name="c", subcore_axis_name="s")

  def body(hp_ref, ce_ref, src_ref, dst_ref, z2_ref, s_out,
           src_b, dst_b, ce_b, rows_b, S_sh,
           sem_in0, sem_in1, sem_g0, sem_g1):
    sem_in = (sem_in0, sem_in1)
    sem_g = (sem_g0, sem_g1)

    cid = lax.axis_index("c")
    sid = lax.axis_index("s")
    hp_c = hp_ref.at[cid]
    ce_c = ce_ref.at[cid]

    base_r = sid * rows_main

    @pl.when(sid < NS - 1)
    def _():
      pltpu.sync_copy(z2_ref.at[pl.ds(0, rows_main)],
                      S_sh.at[pl.ds(base_r, rows_main)])

    @pl.when(sid == NS - 1)
    def _():
      pltpu.sync_copy(z2_ref.at[pl.ds(0, rows_last)],
                      S_sh.at[pl.ds(base_r, rows_last)])

    plsc.subcore_barrier()

    tile_base = sid * epert
    col_base = cid * N

    def issue_linear(c, b):
      base = tile_base + c * CHUNK
      pltpu.async_copy(src_ref.at[pl.ds(base, CHUNK)], src_b.at[b], sem_in[b])
      pltpu.async_copy(dst_ref.at[pl.ds(base, CHUNK)], dst_b.at[b], sem_in[b])
      pltpu.async_copy(ce_c.at[pl.ds(base // 4, CHUNK // 4)], ce_b.at[b],
                       sem_in[b])

    def wait_linear(b):
      pltpu.make_async_copy(src_ref.at[pl.ds(0, CHUNK)], src_b.at[b],
                            sem_in[b]).wait()
      pltpu.make_async_copy(dst_ref.at[pl.ds(0, CHUNK)], dst_b.at[b],
                            sem_in[b]).wait()
      pltpu.make_async_copy(ce_c.at[pl.ds(0, CHUNK // 4)], ce_b.at[b],
                            sem_in[b]).wait()

    def issue_gather(b):
      pltpu.async_copy(hp_c.at[src_b.at[b]], rows_b.at[b], sem_g[b])

    def wait_gather(b):
      pltpu.make_async_copy(hp_c.at[src_b.at[b]], rows_b.at[b],
                            sem_g[b]).wait()

    def process(b):
      def row_body(it, carry):
        for j in range(8):
          rr, half = j // 2, j % 2
          v = (rows_b[b, it * 4 + rr, pl.ds(half * 16, 16)]
               + ce_b[b, it, pl.ds(j * 16, 16)])
          rows_b[b, it * 4 + rr, pl.ds(half * 16, 16)] = jnp.maximum(v, 0.0)
        return carry

      lax.fori_loop(0, CHUNK // 4, row_body, 0)
      pltpu.sync_copy(rows_b.at[b], S_sh.at[dst_b.at[b]], add=True)

    issue_linear(0, 0)
    issue_linear(1, 1)
    wait_linear(0)
    issue_gather(0)

    def pair_body(p, carry):
      c = 2 * p
      wait_linear(1)
      issue_gather(1)
      wait_gather(0)
      process(0)

      @pl.when(c + 2 < iters)
      def _():
        issue_linear(c + 2, 0)

      wait_gather(1)
      process(1)

      @pl.when(c + 3 < iters)
      def _():
        issue_linear(c + 3, 1)

      @pl.when(c + 2 < iters)
      def _():
        wait_linear(0)
        issue_gather(0)
      return carry

    lax.fori_loop(0, iters // 2, pair_body, 0)
    wait_gather(0)
    process(0)
    plsc.subcore_barrier()

    @pl.when(sid < NS - 1)
    def _():
      pltpu.sync_copy(S_sh.at[pl.ds(base_r, rows_main)],
                      s_out.at[pl.ds(col_base + base_r, rows_main)])

    @pl.when(sid == NS - 1)
    def _():
      pltpu.sync_copy(S_sh.at[pl.ds(base_r, rows_last)],
                      s_out.at[pl.ds(col_base + base_r, rows_last)])

  return pl.kernel(
      body,
      mesh=mesh,
      out_type=[jax.ShapeDtypeStruct((2 * N, 32), jnp.float32)],
      compiler_params=pltpu.CompilerParams(use_tc_tiling_on_sc=False,
                                           skip_device_barrier=True),
      scratch_types=[
          pltpu.VMEM((2, CHUNK), jnp.int32),
          pltpu.VMEM((2, CHUNK), jnp.int32),
          pltpu.VMEM((2, CHUNK // 4, 128), jnp.float32),
          pltpu.VMEM((2, CHUNK, 32), jnp.float32),
          pltpu.VMEM_SHARED((N, 32), jnp.float32),
          pltpu.SemaphoreType.DMA,
          pltpu.SemaphoreType.DMA,
          pltpu.SemaphoreType.DMA,
          pltpu.SemaphoreType.DMA,
      ],
  )


def _make_deg_pass(N, E):
  NS = 16
  CK = 40
  half_e = E // 2
  epert = half_e // NS
  iters = epert // CK
  assert epert % CK == 0
  rows_main = ((N // NS) + 7) // 8 * 8
  rows_last = N - rows_main * (NS - 1)

  mesh = plsc.VectorSubcoreMesh(core_axis_name="c", subcore_axis_name="s")

  def body(dst_ref, z1_ref, ones_ref, part_out, dst_b, ones_v, deg_sh,
           sem0, sem1):
    sem = (sem0, sem1)
    cid = lax.axis_index("c")
    sid = lax.axis_index("s")
    pltpu.sync_copy(ones_ref, ones_v)
    base_r = sid * rows_main

    @pl.when(sid < NS - 1)
    def _():
      pltpu.sync_copy(z1_ref.at[pl.ds(0, rows_main)],
                      deg_sh.at[pl.ds(base_r, rows_main)])

    @pl.when(sid == NS - 1)
    def _():
      pltpu.sync_copy(z1_ref.at[pl.ds(0, rows_last)],
                      deg_sh.at[pl.ds(base_r, rows_last)])

    plsc.subcore_barrier()
    tile_base = cid * half_e + sid * epert

    def issue(c, b):
      pltpu.async_copy(dst_ref.at[pl.ds(tile_base + c * CK, CK)],
                       dst_b.at[b], sem[b])

    def wait(b):
      pltpu.make_async_copy(dst_ref.at[pl.ds(0, CK)], dst_b.at[b],
                            sem[b]).wait()

    issue(0, 0)
    issue(1, 1)

    def pair_body(p, carry):
      c = 2 * p
      wait(0)
      pltpu.sync_copy(ones_v, deg_sh.at[dst_b.at[0]], add=True)

      @pl.when(c + 2 < iters)
      def _():
        issue(c + 2, 0)
      wait(1)
      pltpu.sync_copy(ones_v, deg_sh.at[dst_b.at[1]], add=True)

      @pl.when(c + 3 < iters)
      def _():
        issue(c + 3, 1)
      return carry

    lax.fori_loop(0, iters // 2, pair_body, 0)
    wait(0)
    pltpu.sync_copy(ones_v, deg_sh.at[dst_b.at[0]], add=True)
    plsc.subcore_barrier()

    @pl.when(sid < NS - 1)
    def _():
      pltpu.sync_copy(deg_sh.at[pl.ds(base_r, rows_main)],
                      part_out.at[cid, pl.ds(base_r, rows_main)])

    @pl.when(sid == NS - 1)
    def _():
      pltpu.sync_copy(deg_sh.at[pl.ds(base_r, rows_last)],
                      part_out.at[cid, pl.ds(base_r, rows_last)])

  return pl.kernel(
      body,
      mesh=mesh,
      out_type=jax.ShapeDtypeStruct((2, N, 8), jnp.float32),
      compiler_params=pltpu.CompilerParams(use_tc_tiling_on_sc=False,
                                           skip_device_barrier=True),
      scratch_types=[
          pltpu.VMEM((2, 40), jnp.int32),
          pltpu.VMEM((40, 8), jnp.float32),
          pltpu.VMEM_SHARED((N, 8), jnp.float32),
          pltpu.SemaphoreType.DMA,
          pltpu.SemaphoreType.DMA,
      ],
  )


def _encode(x, Wn, bn, A1, BN=2000):
  N = x.shape[0]

  def body(x_ref, wn_ref, bn_ref, a1_ref, h_ref, hp_ref):
    h = jnp.dot(x_ref[...], wn_ref[...],
                preferred_element_type=jnp.float32) + bn_ref[...]
    h_ref[...] = h
    hp = jnp.dot(h, a1_ref[...], preferred_element_type=jnp.float32)
    hp_ref[0] = hp[:, :32]
    hp_ref[1] = hp[:, 32:]

  return pl.pallas_call(
      body,
      grid=(N // BN,),
      in_specs=[
          pl.BlockSpec((BN, x.shape[1]), lambda i: (i, 0)),
          pl.BlockSpec((x.shape[1], H), lambda i: (0, 0)),
          pl.BlockSpec((1, H), lambda i: (0, 0)),
          pl.BlockSpec((H, H), lambda i: (0, 0)),
      ],
      out_specs=[
          pl.BlockSpec((BN, H), lambda i: (i, 0)),
          pl.BlockSpec((2, BN, 32), lambda i: (0, i, 0)),
      ],
      out_shape=[
          jax.ShapeDtypeStruct((N, H), jnp.float32),
          jax.ShapeDtypeStruct((2, N, 32), jnp.float32),
      ],
  )(x, Wn, bn.reshape(1, H), A1)


def _edge_const_layer(ea_p, W_bd, b_pack):
  E4 = ea_p.shape[0]
  BE4 = 5000

  def body(ea_ref, w_ref, b_ref, ce_ref):
    z = jnp.dot(ea_ref[...], w_ref[...],
                preferred_element_type=jnp.float32) + b_ref[...]
    ce_ref[0] = z[:, :128]
    ce_ref[1] = z[:, 128:]

  return pl.pallas_call(
      body,
      grid=(E4 // BE4,),
      in_specs=[
          pl.BlockSpec((BE4, 64), lambda i: (i, 0)),
          pl.BlockSpec((64, 256), lambda i: (0, 0)),
          pl.BlockSpec((1, 256), lambda i: (0, 0)),
      ],
      out_specs=pl.BlockSpec((2, BE4, 128), lambda i: (0, i, 0)),
      out_shape=jax.ShapeDtypeStruct((2, E4, 128), jnp.float32),
  )(ea_p, W_bd, b_pack)


def _update(h, S2, deg, W1, W2, vrow, bu1, u2w, u2b, Anext, BN=2000):
  N = h.shape[0]
  has_next = Anext is not None

  def body(h_ref, s_ref, deg_ref, w1_ref, w2_ref, v_ref, b1_ref, w3_ref,
           b3_ref, *rest):
    if has_next:
      an_ref, hn_ref, hp_ref = rest
    else:
      (hn_ref,) = rest
    hv = h_ref[...]
    w2 = w2_ref[...]
    u = (jnp.dot(hv, w1_ref[...], preferred_element_type=jnp.float32)
         + jnp.dot(s_ref[0], w2[:32, :], preferred_element_type=jnp.float32)
         + jnp.dot(s_ref[1], w2[32:, :], preferred_element_type=jnp.float32)
         + (deg_ref[0][:, :1] + deg_ref[1][:, :1]) * v_ref[...]
         + b1_ref[...])
    u = jnp.maximum(u, 0.0)
    hn = jnp.dot(u, w3_ref[...], preferred_element_type=jnp.float32) + b3_ref[...]
    hn_ref[...] = hn
    if has_next:
      hp = jnp.dot(hn, an_ref[...], preferred_element_type=jnp.float32)
      hp_ref[0] = hp[:, :32]
      hp_ref[1] = hp[:, 32:]

  full = lambda i: (0, 0)
  in_specs = [
      pl.BlockSpec((BN, H), lambda i: (i, 0)),
      pl.BlockSpec((2, BN, 32), lambda i: (0, i, 0)),
      pl.BlockSpec((2, BN, 8), lambda i: (0, i, 0)),
      pl.BlockSpec((H, H), full),
      pl.BlockSpec((H, H), full),
      pl.BlockSpec((1, H), full),
      pl.BlockSpec((1, H), full),
      pl.BlockSpec((H, H), full),
      pl.BlockSpec((1, H), full),
  ]
  out_specs = [pl.BlockSpec((BN, H), lambda i: (i, 0))]
  out_shape = [jax.ShapeDtypeStruct((N, H), jnp.float32)]
  args = [h, S2, deg, W1, W2, vrow.reshape(1, H), bu1.reshape(1, H), u2w,
          u2b.reshape(1, H)]
  if has_next:
    in_specs.append(pl.BlockSpec((H, H), full))
    out_specs.append(pl.BlockSpec((2, BN, 32), lambda i: (0, i, 0)))
    out_shape.append(jax.ShapeDtypeStruct((2, N, 32), jnp.float32))
    args.append(Anext)

  res = pl.pallas_call(
      body,
      grid=(N // BN,),
      in_specs=in_specs,
      out_specs=out_specs,
      out_shape=out_shape,
  )(*args)
  return res if has_next else (res[0], None)


def _pool_readout(h, batch2d, descriptors, r1w, r1b, r2w, r2b, BN=2000):
  N = h.shape[0]
  G, DESC = descriptors.shape
  nblk = N // BN

  def body(h_ref, b_ref, d_ref, r1w_ref, r1b_ref, r2w_ref, r2b_ref, out_ref,
           sums_acc, cnt_acc):
    i = pl.program_id(0)
    onehot = (b_ref[...] == lax.broadcasted_iota(jnp.int32, (1, G), 1)
              ).astype(jnp.float32)
    contrib = lax.dot_general(onehot, h_ref[...], (((0,), (0,)), ((), ())),
                              preferred_element_type=jnp.float32)
    ones_col = jnp.ones((BN, 1), jnp.float32)
    cnt_contrib = lax.dot_general(onehot, ones_col, (((0,), (0,)), ((), ())),
                                  preferred_element_type=jnp.float32)

    @pl.when(i == 0)
    def _():
      sums_acc[...] = contrib
      cnt_acc[...] = cnt_contrib

    @pl.when(i > 0)
    def _():
      sums_acc[...] = sums_acc[...] + contrib
      cnt_acc[...] = cnt_acc[...] + cnt_contrib

    @pl.when(i == nblk - 1)
    def _():
      pooled = sums_acc[...] / jnp.maximum(cnt_acc[...], 1.0)
      r1 = r1w_ref[...]
      z = (jnp.dot(pooled, r1[:H, :], preferred_element_type=jnp.float32)
           + jnp.dot(d_ref[...], r1[H:, :], preferred_element_type=jnp.float32)
           + r1b_ref[...])
      z = jnp.maximum(z, 0.0)
      z2 = jnp.dot(z, r2w_ref[...], preferred_element_type=jnp.float32) \
          + r2b_ref[...]
      out_ref[...] = 1.0 / (1.0 + jnp.exp(-z2))

  full = lambda i: (0, 0)
  return pl.pallas_call(
      body,
      grid=(nblk,),
      in_specs=[
          pl.BlockSpec((BN, H), lambda i: (i, 0)),
          pl.BlockSpec((BN, 1), lambda i: (i, 0)),
          pl.BlockSpec((G, DESC), full),
          pl.BlockSpec((H + DESC, 128), full),
          pl.BlockSpec((1, 128), full),
          pl.BlockSpec((128, 1), full),
          pl.BlockSpec((1, 1), full),
      ],
      out_specs=pl.BlockSpec((G, 1), full),
      out_shape=jax.ShapeDtypeStruct((G, 1), jnp.float32),
      scratch_shapes=[
          pltpu.VMEM((G, H), jnp.float32),
          pltpu.VMEM((G, 1), jnp.float32),
      ],
  )(h, batch2d, descriptors, r1w, r1b.reshape(1, 128), r2w,
    r2b.reshape(1, 1))


def kernel(x, edge_index, edge_attr, batch, descriptors, params):
  N = x.shape[0]
  E = edge_index.shape[1]

  src = edge_index[0].astype(jnp.int32)
  dst = edge_index[1].astype(jnp.int32)
  batch2d = batch.astype(jnp.int32).reshape(N, 1)

  Wn, bn = params["node_enc"]
  We, be = params["edge_enc"]
  layers = params["layers"]

  Bs = [lp["e1"][0][H:] for lp in layers]
  A = [lp["e1"][0][:H] for lp in layers]

  h, hp = _encode(x, Wn, bn, A[0])
  ea_p = edge_attr.reshape(E // 4, 4 * edge_attr.shape[1])
  ce = []
  for B, lp in zip(Bs, layers):
    W = We @ B
    b = be @ B + lp["e1"][1]
    ED = edge_attr.shape[1]
    W_bd = jnp.zeros((4 * ED, 256), jnp.float32)
    for k in range(4):
      W_bd = W_bd.at[k * ED:(k + 1) * ED, k * 32:(k + 1) * 32].set(W[:, :32])
      W_bd = W_bd.at[k * ED:(k + 1) * ED,
                     128 + k * 32:128 + (k + 1) * 32].set(W[:, 32:])
    b_pack = jnp.concatenate([jnp.tile(b[:32], 4), jnp.tile(b[32:], 4)])
    ce.append(_edge_const_layer(ea_p, W_bd, b_pack.reshape(1, 256)))

  z2 = jnp.zeros((3200, 32), jnp.float32)
  z1 = jnp.zeros((3200, 8), jnp.float32)
  ones_in = jnp.ones((40, 8), jnp.float32)

  edge_pass = _make_edge_pass(N, E)
  deg = _make_deg_pass(N, E)(dst, z1, ones_in)

  for l, lp in enumerate(layers):
    res = edge_pass(hp, ce[l], src, dst, z2)
    S2 = res[0] if isinstance(res, (list, tuple)) else res

    u1w, u1b = lp["u1"]
    e2w, e2b = lp["e2"]
    W1 = u1w[:H]
    W2 = e2w @ u1w[H:]
    vrow = e2b @ u1w[H:]
    u2w, u2b = lp["u2"]
    Anext = A[l + 1] if l + 1 < len(layers) else None
    h, hp = _update(h, S2.reshape(2, N, 32), deg, W1, W2, vrow, u1b, u2w,
                    u2b, Anext)

  r1w, r1b = params["r1"]
  r2w, r2b = params["r2"]
  out = _pool_readout(h, batch2d, descriptors, r1w, r1b, r2w, r2b)
  return out.reshape(-1)
